# Initial kernel scaffold; baseline (speedup 1.0000x reference)
import jax
import jax.numpy as jnp
from jax.experimental import pallas as pl


def kernel(x, edge_index, edge_attr, W1, b1, W2, b2, W3, b3, W4, b4):
    raise NotImplementedError("write your pallas kernel here")


# speedup vs baseline: 1.3150x; 1.3150x over previous
import jax
import jax.numpy as jnp
from jax.experimental import pallas as pl

K = 32
D = 16
HID = 128
EPS = 1e-12

NODES = 10000
NB = 200
GRID = NODES // NB


def _acos(c):
    ax = jnp.abs(c)
    p = jnp.float32(-0.0012624911)
    p = p * ax + jnp.float32(0.0066700901)
    p = p * ax + jnp.float32(-0.0170881256)
    p = p * ax + jnp.float32(0.0308918810)
    p = p * ax + jnp.float32(-0.0501743046)
    p = p * ax + jnp.float32(0.0889789874)
    p = p * ax + jnp.float32(-0.2145988016)
    p = p * ax + jnp.float32(1.5707963050)
    r = jnp.sqrt(jnp.maximum(1.0 - ax, 0.0)) * p
    return jnp.where(c >= 0, r, jnp.float32(3.14159265358979) - r)


def _fused_kernel(e_ref, w1e_ref, b1_ref, w2_ref, b2_ref, w3_ref, b3_ref,
                  w4_ref, b4_ref, o_ref):
    e = e_ref[...]
    er = jnp.roll(e, -1, axis=0)
    sq1 = jnp.sum(e * e, axis=1, keepdims=True) + EPS
    sq2 = jnp.sum(er * er, axis=1, keepdims=True) + EPS
    dot = jnp.sum(e * er, axis=1, keepdims=True)
    c = dot * jax.lax.rsqrt(sq1 * sq2)
    c = jnp.clip(c, -1.0, 1.0)
    ang = _acos(c)
    ang3 = ang.reshape(NB, K, 1)
    h = jnp.sum(ang3 * w1e_ref[...][None], axis=1) + b1_ref[...]
    h = jnp.tanh(h)
    h = jnp.tanh(jnp.dot(h, w2_ref[...],
                         preferred_element_type=jnp.float32) + b2_ref[...])
    h = jnp.tanh(jnp.dot(h, w3_ref[...],
                         preferred_element_type=jnp.float32) + b3_ref[...])
    o = jax.nn.sigmoid(jnp.dot(h, w4_ref[...],
                               preferred_element_type=jnp.float32) + b4_ref[...])
    o_ref[...] = o


def kernel(x, edge_index, edge_attr, W1, b1, W2, b2, W3, b3, W4, b4):
    del x, edge_index
    W1e = jnp.zeros((K, HID), jnp.float32).at[0::2].set(W1)
    out = pl.pallas_call(
        _fused_kernel,
        grid=(GRID,),
        in_specs=[
            pl.BlockSpec((NB * K, D), lambda i: (i, 0)),
            pl.BlockSpec((K, HID), lambda i: (0, 0)),
            pl.BlockSpec((1, HID), lambda i: (0, 0)),
            pl.BlockSpec((HID, HID), lambda i: (0, 0)),
            pl.BlockSpec((1, HID), lambda i: (0, 0)),
            pl.BlockSpec((HID, HID), lambda i: (0, 0)),
            pl.BlockSpec((1, HID), lambda i: (0, 0)),
            pl.BlockSpec((HID, 1), lambda i: (0, 0)),
            pl.BlockSpec((1, 1), lambda i: (0, 0)),
        ],
        out_specs=pl.BlockSpec((NB, 1), lambda i: (i, 0)),
        out_shape=jax.ShapeDtypeStruct((NODES, 1), jnp.float32),
    )(edge_attr, W1e, b1.reshape(1, HID), W2, b2.reshape(1, HID),
      W3, b3.reshape(1, HID), W4, b4.reshape(1, 1))
    return out[:, 0]


# speedup vs baseline: 2.2466x; 1.7084x over previous
import jax
import jax.numpy as jnp
from jax.experimental import pallas as pl

K = 32
D = 16
F = K * D
HID = 128
EPS = 1e-12

NODES = 10000
NPAD = 10240
NL = 1024
GRID = NPAD // NL


def _acos(c):
    ax = jnp.abs(c)
    p = jnp.float32(-0.0012624911)
    p = p * ax + jnp.float32(0.0066700901)
    p = p * ax + jnp.float32(-0.0170881256)
    p = p * ax + jnp.float32(0.0308918810)
    p = p * ax + jnp.float32(-0.0501743046)
    p = p * ax + jnp.float32(0.0889789874)
    p = p * ax + jnp.float32(-0.2145988016)
    p = p * ax + jnp.float32(1.5707963050)
    r = jnp.sqrt(jnp.maximum(1.0 - ax, 0.0)) * p
    return jnp.where(c >= 0, r, jnp.float32(3.14159265358979) - r)


def _fused_kernel(t_ref, w1_ref, b1_ref, w2_ref, b2_ref, w3_ref, b3_ref,
                  w4_ref, b4_ref, o_ref):
    t = t_ref[...]
    ts = jnp.roll(t, -D, axis=0)
    sq = jnp.sum((t * t).reshape(K, D, NL), axis=1) + EPS
    dt = jnp.sum((t * ts).reshape(K, D, NL), axis=1)
    sq2 = jnp.roll(sq, -1, axis=0)
    c = dt * jax.lax.rsqrt(sq * sq2)
    c = jnp.clip(c, -1.0, 1.0)
    ang = _acos(c)
    h = jnp.tanh(jnp.dot(w1_ref[...], ang,
                         preferred_element_type=jnp.float32) + b1_ref[...])
    h = jnp.tanh(jnp.dot(w2_ref[...], h,
                         preferred_element_type=jnp.float32) + b2_ref[...])
    h = jnp.tanh(jnp.dot(w3_ref[...], h,
                         preferred_element_type=jnp.float32) + b3_ref[...])
    o = jax.nn.sigmoid(jnp.dot(w4_ref[...], h,
                               preferred_element_type=jnp.float32) + b4_ref[...])
    o_ref[...] = o


def kernel(x, edge_index, edge_attr, W1, b1, W2, b2, W3, b3, W4, b4):
    del x, edge_index
    ea = edge_attr.reshape(NODES, F)
    ea = jnp.pad(ea, ((0, NPAD - NODES), (0, 0)))
    t_full = ea.T
    W1eT = jnp.zeros((HID, K), jnp.float32).at[:, 0::2].set(W1.T)
    out = pl.pallas_call(
        _fused_kernel,
        grid=(GRID,),
        in_specs=[
            pl.BlockSpec((F, NL), lambda i: (0, i)),
            pl.BlockSpec((HID, K), lambda i: (0, 0)),
            pl.BlockSpec((HID, 1), lambda i: (0, 0)),
            pl.BlockSpec((HID, HID), lambda i: (0, 0)),
            pl.BlockSpec((HID, 1), lambda i: (0, 0)),
            pl.BlockSpec((HID, HID), lambda i: (0, 0)),
            pl.BlockSpec((HID, 1), lambda i: (0, 0)),
            pl.BlockSpec((1, HID), lambda i: (0, 0)),
            pl.BlockSpec((1, 1), lambda i: (0, 0)),
        ],
        out_specs=pl.BlockSpec((1, NL), lambda i: (0, i)),
        out_shape=jax.ShapeDtypeStruct((1, NPAD), jnp.float32),
    )(t_full, W1eT, b1.reshape(HID, 1), W2.T, b2.reshape(HID, 1),
      W3.T, b3.reshape(HID, 1), W4.T, b4.reshape(1, 1))
    return out[0, :NODES]


# speedup vs baseline: 2.4672x; 1.0982x over previous
import jax
import jax.numpy as jnp
from jax.experimental import pallas as pl

K = 32
D = 16
F = K * D
HID = 128
EPS = 1e-12

NODES = 10000
NPAD = 10240
NN = 1024
GRID = NPAD // NN


def _acos(c):
    ax = jnp.abs(c)
    p = jnp.float32(-0.0012624911)
    p = p * ax + jnp.float32(0.0066700901)
    p = p * ax + jnp.float32(-0.0170881256)
    p = p * ax + jnp.float32(0.0308918810)
    p = p * ax + jnp.float32(-0.0501743046)
    p = p * ax + jnp.float32(0.0889789874)
    p = p * ax + jnp.float32(-0.2145988016)
    p = p * ax + jnp.float32(1.5707963050)
    r = jnp.sqrt(jnp.maximum(1.0 - ax, 0.0)) * p
    return jnp.where(c >= 0, r, jnp.float32(3.14159265358979) - r)


def _fused_kernel(t_ref, sel_ref, w1_ref, b1_ref, w2_ref, b2_ref,
                  w3_ref, b3_ref, w4_ref, b4_ref, o_ref):
    t = t_ref[...]
    tr = jnp.roll(t, -D, axis=1)
    sel = sel_ref[...]
    sq = jnp.dot(t * t, sel,
                 preferred_element_type=jnp.float32) + EPS
    dt = jnp.dot(t * tr, sel,
                 preferred_element_type=jnp.float32)
    sq2 = jnp.roll(sq, -1, axis=1)
    c = dt * jax.lax.rsqrt(sq * sq2)
    c = jnp.clip(c, -1.0, 1.0)
    ang = _acos(c)
    h = jnp.tanh(jnp.dot(ang, w1_ref[...],
                         preferred_element_type=jnp.float32) + b1_ref[...])
    h = jnp.tanh(jnp.dot(h, w2_ref[...],
                         preferred_element_type=jnp.float32) + b2_ref[...])
    h = jnp.tanh(jnp.dot(h, w3_ref[...],
                         preferred_element_type=jnp.float32) + b3_ref[...])
    o = jax.nn.sigmoid(jnp.dot(h, w4_ref[...],
                               preferred_element_type=jnp.float32) + b4_ref[...])
    o_ref[...] = o


def kernel(x, edge_index, edge_attr, W1, b1, W2, b2, W3, b3, W4, b4):
    del x, edge_index
    ea = edge_attr.reshape(NODES, F)
    ea = jnp.pad(ea, ((0, NPAD - NODES), (0, 0)))
    sel = (jax.lax.broadcasted_iota(jnp.int32, (F, K), 0) // D ==
           jax.lax.broadcasted_iota(jnp.int32, (F, K), 1)).astype(jnp.float32)
    W1e = jnp.zeros((K, HID), jnp.float32).at[0::2].set(W1)
    out = pl.pallas_call(
        _fused_kernel,
        grid=(GRID,),
        in_specs=[
            pl.BlockSpec((NN, F), lambda i: (i, 0)),
            pl.BlockSpec((F, K), lambda i: (0, 0)),
            pl.BlockSpec((K, HID), lambda i: (0, 0)),
            pl.BlockSpec((1, HID), lambda i: (0, 0)),
            pl.BlockSpec((HID, HID), lambda i: (0, 0)),
            pl.BlockSpec((1, HID), lambda i: (0, 0)),
            pl.BlockSpec((HID, HID), lambda i: (0, 0)),
            pl.BlockSpec((1, HID), lambda i: (0, 0)),
            pl.BlockSpec((HID, 1), lambda i: (0, 0)),
            pl.BlockSpec((1, 1), lambda i: (0, 0)),
        ],
        out_specs=pl.BlockSpec((NN, 1), lambda i: (i, 0)),
        out_shape=jax.ShapeDtypeStruct((NPAD, 1), jnp.float32),
    )(ea, sel, W1e, b1.reshape(1, HID), W2, b2.reshape(1, HID),
      W3, b3.reshape(1, HID), W4, b4.reshape(1, 1))
    return out[:NODES, 0]


# speedup vs baseline: 2.6464x; 1.0726x over previous
import jax
import jax.numpy as jnp
from jax.experimental import pallas as pl

K = 32
D = 16
F = K * D
HID = 128
EPS = 1e-12

NODES = 10000
NN = 1024
GRID = -(-NODES // NN)


def _acos(c):
    ax = jnp.abs(c)
    p = jnp.float32(-0.0012624911)
    p = p * ax + jnp.float32(0.0066700901)
    p = p * ax + jnp.float32(-0.0170881256)
    p = p * ax + jnp.float32(0.0308918810)
    p = p * ax + jnp.float32(-0.0501743046)
    p = p * ax + jnp.float32(0.0889789874)
    p = p * ax + jnp.float32(-0.2145988016)
    p = p * ax + jnp.float32(1.5707963050)
    r = jnp.sqrt(jnp.maximum(1.0 - ax, 0.0)) * p
    return jnp.where(c >= 0, r, jnp.float32(3.14159265358979) - r)


def _fused_kernel(t_ref, sel_ref, w1_ref, b1_ref, w2_ref, b2_ref,
                  w3_ref, b3_ref, w4_ref, b4_ref, o_ref):
    t = t_ref[...]
    tr = jnp.roll(t, -D, axis=1)
    sel = sel_ref[...]
    sq = jnp.dot(t * t, sel,
                 preferred_element_type=jnp.float32) + EPS
    dt = jnp.dot(t * tr, sel,
                 preferred_element_type=jnp.float32)
    sq2 = jnp.roll(sq, -1, axis=1)
    c = dt * jax.lax.rsqrt(sq * sq2)
    c = jnp.clip(c, -1.0, 1.0)
    ang = _acos(c)
    h = jnp.tanh(jnp.dot(ang, w1_ref[...],
                         preferred_element_type=jnp.float32) + b1_ref[...])
    h = jnp.tanh(jnp.dot(h, w2_ref[...],
                         preferred_element_type=jnp.float32) + b2_ref[...])
    h = jnp.tanh(jnp.dot(h, w3_ref[...],
                         preferred_element_type=jnp.float32) + b3_ref[...])
    o = jax.nn.sigmoid(jnp.dot(h, w4_ref[...],
                               preferred_element_type=jnp.float32) + b4_ref[...])
    o_ref[...] = o


def kernel(x, edge_index, edge_attr, W1, b1, W2, b2, W3, b3, W4, b4):
    del x, edge_index
    ea = edge_attr.reshape(NODES, F)
    sel =(jax.lax.broadcasted_iota(jnp.int32, (F, K), 0) // D ==
           jax.lax.broadcasted_iota(jnp.int32, (F, K), 1)).astype(jnp.float32)
    W1e = jnp.zeros((K, HID), jnp.float32).at[0::2].set(W1)
    out = pl.pallas_call(
        _fused_kernel,
        grid=(GRID,),
        in_specs=[
            pl.BlockSpec((NN, F), lambda i: (i, 0)),
            pl.BlockSpec((F, K), lambda i: (0, 0)),
            pl.BlockSpec((K, HID), lambda i: (0, 0)),
            pl.BlockSpec((1, HID), lambda i: (0, 0)),
            pl.BlockSpec((HID, HID), lambda i: (0, 0)),
            pl.BlockSpec((1, HID), lambda i: (0, 0)),
            pl.BlockSpec((HID, HID), lambda i: (0, 0)),
            pl.BlockSpec((1, HID), lambda i: (0, 0)),
            pl.BlockSpec((HID, 1), lambda i: (0, 0)),
            pl.BlockSpec((1, 1), lambda i: (0, 0)),
        ],
        out_specs=pl.BlockSpec((NN, 1), lambda i: (i, 0)),
        out_shape=jax.ShapeDtypeStruct((NODES, 1), jnp.float32),
    )(ea, sel, W1e, b1.reshape(1, HID), W2, b2.reshape(1, HID),
      W3, b3.reshape(1, HID), W4, b4.reshape(1, 1))
    return out[:, 0]
